# baseline (speedup 1.0000x reference)
import functools

import jax
import jax.numpy as jnp
from jax import lax
from jax.experimental import pallas as pl
from jax.experimental.pallas import tpu as pltpu
from jax.experimental.pallas import tpu_sc as plsc

VOCAB = 100000
BASIC_DIM = 300
MODIF_DIM = 100
FUSED_DIM = 400
BATCH = 16384
HIST = 50

_info = plsc.get_sparse_core_info()
NC = _info.num_cores
NS = _info.num_subcores
NW = NC * NS
SLABS_W = BATCH // NW
NBUF = 4
KB = 64
NBLK = SLABS_W // KB

_mesh = plsc.VectorSubcoreMesh(core_axis_name="c", subcore_axis_name="s")


@functools.partial(
    pl.kernel,
    mesh=_mesh,
    compiler_params=pltpu.CompilerParams(use_tc_tiling_on_sc=False),
    out_type=jax.ShapeDtypeStruct((BATCH, HIST, FUSED_DIM), jnp.float32),
    scratch_types=[
        pltpu.VMEM((KB, HIST), jnp.int32),
        pltpu.VMEM((NBUF, HIST, FUSED_DIM), jnp.float32),
        pltpu.SemaphoreType.DMA,
        pltpu.SemaphoreType.DMA,
        pltpu.SemaphoreType.DMA,
        pltpu.SemaphoreType.DMA,
        pltpu.SemaphoreType.DMA,
    ],
)
def _gather(table_hbm, ids_hbm, out_hbm, idx_v, rows_v,
            sem_g, sw0, sw1, sw2, sw3):
    w = lax.axis_index("s") * NC + lax.axis_index("c")
    base = w * SLABS_W
    sems_w = (sw0, sw1, sw2, sw3)

    def slab_step(s, bi):
        b = base + s
        @pl.when(s >= NBUF)
        def _():
            pltpu.make_async_copy(
                rows_v.at[bi], out_hbm.at[b - NBUF], sems_w[bi]).wait()

        pltpu.async_copy(
            table_hbm.at[idx_v.at[s % KB]], rows_v.at[bi], sem_g).wait()
        pltpu.async_copy(rows_v.at[bi], out_hbm.at[b], sems_w[bi])

    def inner(i, blk):
        for r in range(NBUF):
            slab_step(blk * KB + i * NBUF + r, r)
        return blk

    def block(blk, carry):
        pltpu.sync_copy(ids_hbm.at[pl.ds(base + blk * KB, KB)], idx_v)
        lax.fori_loop(0, KB // NBUF, inner, blk, unroll=False)
        return carry

    lax.fori_loop(0, NBLK, block, 0, unroll=False)

    for r in range(NBUF):
        b = base + SLABS_W - NBUF + r
        pltpu.make_async_copy(
            rows_v.at[r], out_hbm.at[b], sems_w[r]).wait()


def kernel(word_ids, basic, modif):
    fused = jnp.concatenate([basic, modif], axis=1)
    return _gather(fused, word_ids)
